# Initial kernel scaffold; baseline (speedup 1.0000x reference)
import jax
import jax.numpy as jnp
from jax.experimental import pallas as pl


def kernel(x, w1, b1, w2, b2, w3, b3, w4, b4):
    raise NotImplementedError("write your pallas kernel here")


# speedup vs baseline: 1.6322x; 1.6322x over previous
import jax
import jax.numpy as jnp
from jax.experimental import pallas as pl
from jax.experimental.pallas import tpu as pltpu

_DIMS = (29, 256, 64, 32, 30)


def _mlp_kernel(x_ref, w1_ref, b1_ref, w2_ref, b2_ref, w3_ref, b3_ref,
                w4_ref, b4_ref, o_ref):
    h = x_ref[...].astype(jnp.bfloat16)

    def layer(h, w_ref, b_ref):
        y = jnp.dot(h, w_ref[...], preferred_element_type=jnp.float32)
        return jnp.maximum(y + b_ref[...], 0.0)

    h = layer(h, w1_ref, b1_ref).astype(jnp.bfloat16)
    h = layer(h, w2_ref, b2_ref).astype(jnp.bfloat16)
    h = layer(h, w3_ref, b3_ref).astype(jnp.bfloat16)
    o_ref[...] = layer(h, w4_ref, b4_ref)


def kernel(x, w1, b1, w2, b2, w3, b3, w4, b4):
    batch, in_dim = x.shape
    assert in_dim == _DIMS[0]

    tb = 1024
    assert batch % tb == 0
    grid = (batch // tb,)

    ws = [w.astype(jnp.bfloat16) for w in (w1, w2, w3, w4)]
    bs = [b.reshape(1, -1) for b in (b1, b2, b3, b4)]

    x_spec = pl.BlockSpec((tb, in_dim), lambda i: (i, 0))
    out_spec = pl.BlockSpec((tb, _DIMS[-1]), lambda i: (i, 0))
    param_specs = []
    for w, b in zip(ws, bs):
        param_specs.append(pl.BlockSpec(w.shape, lambda i: (0, 0)))
        param_specs.append(pl.BlockSpec(b.shape, lambda i: (0, 0)))

    args = [x]
    for w, b in zip(ws, bs):
        args.extend([w, b])

    flops = 2 * batch * sum(_DIMS[i] * _DIMS[i + 1] for i in range(4))
    bytes_accessed = 4 * batch * (_DIMS[0] + _DIMS[-1])

    return pl.pallas_call(
        _mlp_kernel,
        out_shape=jax.ShapeDtypeStruct((batch, _DIMS[-1]), jnp.float32),
        grid=grid,
        in_specs=[x_spec] + param_specs,
        out_specs=out_spec,
        compiler_params=pltpu.CompilerParams(
            dimension_semantics=("parallel",)),
        cost_estimate=pl.CostEstimate(flops=flops, transcendentals=0,
                                      bytes_accessed=bytes_accessed),
    )(*args)


# speedup vs baseline: 2.3209x; 1.4220x over previous
import jax
import jax.numpy as jnp
from jax.experimental import pallas as pl
from jax.experimental.pallas import tpu as pltpu

_DIMS = (29, 256, 64, 32, 30)


def _mlp_kernel(x_ref, w1_ref, b1_ref, w2_ref, b2_ref, w3_ref, b3_ref,
                w4_ref, b4_ref, o_ref):
    h = x_ref[...].astype(jnp.bfloat16)

    def layer(h, w_ref, b_ref):
        y = jnp.dot(h, w_ref[...], preferred_element_type=jnp.float32)
        return jnp.maximum(y + b_ref[...], 0.0)

    h = layer(h, w1_ref, b1_ref).astype(jnp.bfloat16)
    h = layer(h, w2_ref, b2_ref).astype(jnp.bfloat16)
    h = layer(h, w3_ref, b3_ref).astype(jnp.bfloat16)
    o_ref[...] = layer(h, w4_ref, b4_ref)


def kernel(x, w1, b1, w2, b2, w3, b3, w4, b4):
    batch, in_dim = x.shape
    assert in_dim == _DIMS[0]

    tb = 4096
    assert batch % tb == 0
    grid = (batch // tb,)

    ws = [w.astype(jnp.bfloat16) for w in (w1, w2, w3, w4)]
    bs = [b.reshape(1, -1) for b in (b1, b2, b3, b4)]

    x_spec = pl.BlockSpec((tb, in_dim), lambda i: (i, 0))
    out_spec = pl.BlockSpec((tb, _DIMS[-1]), lambda i: (i, 0))
    param_specs = []
    for w, b in zip(ws, bs):
        param_specs.append(pl.BlockSpec(w.shape, lambda i: (0, 0)))
        param_specs.append(pl.BlockSpec(b.shape, lambda i: (0, 0)))

    args = [x]
    for w, b in zip(ws, bs):
        args.extend([w, b])

    flops = 2 * batch * sum(_DIMS[i] * _DIMS[i + 1] for i in range(4))
    bytes_accessed = 4 * batch * (_DIMS[0] + _DIMS[-1])

    return pl.pallas_call(
        _mlp_kernel,
        out_shape=jax.ShapeDtypeStruct((batch, _DIMS[-1]), jnp.float32),
        grid=grid,
        in_specs=[x_spec] + param_specs,
        out_specs=out_spec,
        compiler_params=pltpu.CompilerParams(
            dimension_semantics=("parallel",)),
        cost_estimate=pl.CostEstimate(flops=flops, transcendentals=0,
                                      bytes_accessed=bytes_accessed),
    )(*args)
